# Initial kernel scaffold; baseline (speedup 1.0000x reference)
import jax
import jax.numpy as jnp
from jax.experimental import pallas as pl


def kernel(img, flo):
    raise NotImplementedError("write your pallas kernel here")


# speedup vs baseline: 51.0691x; 51.0691x over previous
import jax
import jax.numpy as jnp
from jax import lax
from jax.experimental import pallas as pl
from jax.experimental.pallas import tpu as pltpu
from jax.experimental.pallas import tpu_sc as plsc

N, C, H, W = 8, 3, 512, 512
HHALF = H // 2
NREAL = HHALF * W
TRASH = 512
NSLOTS = NREAL + TRASH
NWORDS = C * NSLOTS
WORDS_PER_TILE = NWORDS // 16
ROWS_PER_TILE = 32
STAGE = 16
CHUNK_ROWS = 2
CHUNK_PX = CHUNK_ROWS * W
NPLANE = 12
NSUB = NPLANE * (CHUNK_PX // 128)
DMAGRP = 24
ZROWS = 8192
EXT_ROWS = HHALF // 16


def _body(img_hbm, flo_hbm, out_hbm, img_v, flo_v, idx_v, val_v, zero_v,
          ext_out, sp, sem):
    cid = lax.axis_index("c")
    sid = lax.axis_index("s")
    iota = lax.iota(jnp.int32, 16)

    zf = jnp.zeros((16,), jnp.float32)

    def _fill(k, carry):
        zero_v[pl.ds(k * 16, 16)] = zf
        return carry

    lax.fori_loop(0, ZROWS // 16, _fill, 0)

    row0_tile = sid * ROWS_PER_TILE
    trash0 = NREAL + sid * 32 + iota * 2
    zbase = sid * WORDS_PER_TILE

    def _pass(q, carry):
        b = q >> 1
        p = q & 1
        n = cid * 4 + b
        prow0 = p * HHALF

        for j in range(3):
            pltpu.sync_copy(zero_v, sp.at[pl.ds(zbase + j * ZROWS, ZROWS)])
        pltpu.sync_copy(zero_v.at[pl.ds(0, 96)],
                        sp.at[pl.ds(zbase + 3 * ZROWS, 96)])
        plsc.subcore_barrier()

        def _stage(h, carry2):
            row0 = row0_tile + h * STAGE
            pltpu.sync_copy(img_hbm.at[n, :, pl.ds(row0, STAGE), :], img_v)
            pltpu.sync_copy(flo_hbm.at[n, :, pl.ds(row0, STAGE), :], flo_v)

            def _chunk(ch, carry3):
                def _compute(i, carry4):
                    lr = ch * CHUNK_ROWS + (i >> 5)
                    col0 = (i & 31) * 16
                    y = flo_v[0, lr, pl.ds(col0, 16)]
                    x = flo_v[1, lr, pl.ds(col0, 16)]
                    xc = jnp.minimum(jnp.maximum(x, -2048.0), 2047.0)
                    yc = jnp.minimum(jnp.maximum(y, -2048.0), 2047.0)
                    x1i = xc.astype(jnp.int32)
                    x1i = x1i - (x1i.astype(jnp.float32) > xc).astype(jnp.int32)
                    y1i = yc.astype(jnp.int32)
                    y1i = y1i - (y1i.astype(jnp.float32) > yc).astype(jnp.int32)
                    x1f = x1i.astype(jnp.float32)
                    y1f = y1i.astype(jnp.float32)
                    wx1 = (x1f + 1.0) - xc
                    wx2 = xc - x1f
                    wy1 = (y1f + 1.0) - yc
                    wy2 = yc - y1f
                    r = x1i + (row0 + lr) - prow0
                    colv = y1i + col0 + iota
                    ws = (wx1 * wy1, wx1 * wy2, wx2 * wy1, wx2 * wy2)
                    imgs = tuple(img_v[c, lr, pl.ds(col0, 16)]
                                 for c in range(C))
                    subrow = i >> 3
                    scol = (i & 7) * 16
                    for d in range(4):
                        tr = r + (d >> 1)
                        tc = colv + (d & 1)
                        valid = ((tr >= 0) & (tr < HHALF) &
                                 (tc >= 0) & (tc < W))
                        slot = jnp.where(valid, tr * W + tc,
                                         trash0 + (d & 1))
                        for c in range(C):
                            krow = (d * C + c) * 8 + subrow
                            idx_v[krow, pl.ds(scol, 16)] = slot + c * NSLOTS
                            val_v[krow, pl.ds(scol, 16)] = imgs[c] * ws[d]
                    return carry4

                lax.fori_loop(0, CHUNK_PX // 16, _compute, 0)

                for g in range(NSUB // DMAGRP):
                    copies = []
                    for m in range(g * DMAGRP, (g + 1) * DMAGRP):
                        copies.append(pltpu.async_copy(
                            val_v.at[m], sp.at[idx_v.at[m]], sem, add=True))
                    for cp in copies:
                        cp.wait()
                return carry3

            lax.fori_loop(0, STAGE // CHUNK_ROWS, _chunk, 0)
            return carry2

        lax.fori_loop(0, ROWS_PER_TILE // STAGE, _stage, 0)
        plsc.subcore_barrier()

        def _extract(rl, carry2):
            rh = sid * EXT_ROWS + rl
            for c in range(C):
                pltpu.sync_copy(sp.at[pl.ds(c * NSLOTS + rh * W, W)],
                                ext_out.at[c])
            pltpu.sync_copy(ext_out, out_hbm.at[n, :, prow0 + rh, :])
            return carry2

        lax.fori_loop(0, EXT_ROWS, _extract, 0)
        plsc.subcore_barrier()
        return carry

    lax.fori_loop(0, 8, _pass, 0)


@jax.jit
def kernel(img, flo):
    mesh = plsc.VectorSubcoreMesh(core_axis_name="c", subcore_axis_name="s")
    fwd = pl.kernel(
        _body,
        out_type=jax.ShapeDtypeStruct((N, C, H, W), jnp.float32),
        mesh=mesh,
        compiler_params=pltpu.CompilerParams(
            needs_layout_passes=False, use_tc_tiling_on_sc=False),
        scratch_types=[
            pltpu.VMEM((C, STAGE, W), jnp.float32),
            pltpu.VMEM((2, STAGE, W), jnp.float32),
            pltpu.VMEM((NSUB, 128), jnp.int32),
            pltpu.VMEM((NSUB, 128), jnp.float32),
            pltpu.VMEM((ZROWS,), jnp.float32),
            pltpu.VMEM((C, W), jnp.float32),
            pltpu.VMEM_SHARED((NWORDS,), jnp.float32),
            pltpu.SemaphoreType.DMA,
        ],
    )
    return fwd(img, flo)


# speedup vs baseline: 58.4687x; 1.1449x over previous
import jax
import jax.numpy as jnp
from jax import lax
from jax.experimental import pallas as pl
from jax.experimental.pallas import tpu as pltpu
from jax.experimental.pallas import tpu_sc as plsc

N, C, H, W = 8, 3, 512, 512
HHALF = H // 2
NREAL = HHALF * W
TRASH = 512
NSLOTS = NREAL + TRASH
NWORDS = C * NSLOTS
WORDS_PER_TILE = NWORDS // 16
ROWS_PER_TILE = 32
STAGE = 16
CHUNK_ROWS = 2
CHUNK_PX = CHUNK_ROWS * W
NPLANE = 12
NSUB = NPLANE * (CHUNK_PX // 128)
DMAGRP = 24
ZROWS = 8192
EXT_ROWS = HHALF // 16


def _body(img_hbm, flo_hbm, out_hbm, img_v, flo_v, idx_a, val_a, idx_b,
          val_b, zero_v, ext_out, sp, sem):
    cid = lax.axis_index("c")
    sid = lax.axis_index("s")
    iota = lax.iota(jnp.int32, 16)

    zf = jnp.zeros((16,), jnp.float32)

    def _fill(k, carry):
        zero_v[pl.ds(k * 16, 16)] = zf
        return carry

    lax.fori_loop(0, ZROWS // 16, _fill, 0)

    row0_tile = sid * ROWS_PER_TILE
    trash0 = NREAL + sid * 32 + iota * 2
    zbase = sid * WORDS_PER_TILE

    def _pass(q, carry):
        b = q >> 1
        p = q & 1
        n = cid * 4 + b
        prow0 = p * HHALF

        for j in range(3):
            pltpu.sync_copy(zero_v, sp.at[pl.ds(zbase + j * ZROWS, ZROWS)])
        pltpu.sync_copy(zero_v.at[pl.ds(0, 96)],
                        sp.at[pl.ds(zbase + 3 * ZROWS, 96)])
        plsc.subcore_barrier()

        def _stage(h, carry2):
            row0 = row0_tile + h * STAGE
            pltpu.sync_copy(img_hbm.at[n, :, pl.ds(row0, STAGE), :], img_v)
            pltpu.sync_copy(flo_hbm.at[n, :, pl.ds(row0, STAGE), :], flo_v)

            def _make_compute(ch, idx_v, val_v):
                def _compute(i, carry4):
                    lr = ch * CHUNK_ROWS + (i >> 5)
                    col0 = (i & 31) * 16
                    y = flo_v[0, lr, pl.ds(col0, 16)]
                    x = flo_v[1, lr, pl.ds(col0, 16)]
                    xc = jnp.minimum(jnp.maximum(x, -2048.0), 2047.0)
                    yc = jnp.minimum(jnp.maximum(y, -2048.0), 2047.0)
                    x1i = xc.astype(jnp.int32)
                    x1i = x1i - (x1i.astype(jnp.float32) > xc).astype(jnp.int32)
                    y1i = yc.astype(jnp.int32)
                    y1i = y1i - (y1i.astype(jnp.float32) > yc).astype(jnp.int32)
                    x1f = x1i.astype(jnp.float32)
                    y1f = y1i.astype(jnp.float32)
                    wx1 = (x1f + 1.0) - xc
                    wx2 = xc - x1f
                    wy1 = (y1f + 1.0) - yc
                    wy2 = yc - y1f
                    r = x1i + (row0 + lr) - prow0
                    colv = y1i + col0 + iota
                    ws = (wx1 * wy1, wx1 * wy2, wx2 * wy1, wx2 * wy2)
                    imgs = tuple(img_v[c, lr, pl.ds(col0, 16)]
                                 for c in range(C))
                    subrow = i >> 3
                    scol = (i & 7) * 16
                    for d in range(4):
                        tr = r + (d >> 1)
                        tc = colv + (d & 1)
                        valid = ((tr >= 0) & (tr < HHALF) &
                                 (tc >= 0) & (tc < W))
                        slot = jnp.where(valid, tr * W + tc,
                                         trash0 + (d & 1))
                        for c in range(C):
                            krow = (d * C + c) * 8 + subrow
                            idx_v[krow, pl.ds(scol, 16)] = slot + c * NSLOTS
                            val_v[krow, pl.ds(scol, 16)] = imgs[c] * ws[d]
                    return carry4

                return _compute

            prev = None
            for ch in range(STAGE // CHUNK_ROWS):
                idx_v, val_v = ((idx_a, val_a), (idx_b, val_b))[ch & 1]
                lax.fori_loop(0, CHUNK_PX // 16,
                              _make_compute(ch, idx_v, val_v), 0)
                if prev is not None:
                    for cp in prev:
                        cp.wait()
                prev = [pltpu.async_copy(val_v.at[m], sp.at[idx_v.at[m]],
                                         sem, add=True)
                        for m in range(NSUB)]
            for cp in prev:
                cp.wait()
            return carry2

        lax.fori_loop(0, ROWS_PER_TILE // STAGE, _stage, 0)
        plsc.subcore_barrier()

        def _extract(rl, carry2):
            rh = sid * EXT_ROWS + rl
            for c in range(C):
                pltpu.sync_copy(sp.at[pl.ds(c * NSLOTS + rh * W, W)],
                                ext_out.at[c])
            pltpu.sync_copy(ext_out, out_hbm.at[n, :, prow0 + rh, :])
            return carry2

        lax.fori_loop(0, EXT_ROWS, _extract, 0)
        plsc.subcore_barrier()
        return carry

    lax.fori_loop(0, 8, _pass, 0)


@jax.jit
def kernel(img, flo):
    mesh = plsc.VectorSubcoreMesh(core_axis_name="c", subcore_axis_name="s")
    fwd = pl.kernel(
        _body,
        out_type=jax.ShapeDtypeStruct((N, C, H, W), jnp.float32),
        mesh=mesh,
        compiler_params=pltpu.CompilerParams(
            needs_layout_passes=False, use_tc_tiling_on_sc=False),
        scratch_types=[
            pltpu.VMEM((C, STAGE, W), jnp.float32),
            pltpu.VMEM((2, STAGE, W), jnp.float32),
            pltpu.VMEM((NSUB, 128), jnp.int32),
            pltpu.VMEM((NSUB, 128), jnp.float32),
            pltpu.VMEM((NSUB, 128), jnp.int32),
            pltpu.VMEM((NSUB, 128), jnp.float32),
            pltpu.VMEM((ZROWS,), jnp.float32),
            pltpu.VMEM((C, W), jnp.float32),
            pltpu.VMEM_SHARED((NWORDS,), jnp.float32),
            pltpu.SemaphoreType.DMA,
        ],
    )
    return fwd(img, flo)
